# baseline (speedup 1.0000x reference)
import functools
import math

import jax
import jax.numpy as jnp
from jax import lax
from jax.experimental import pallas as pl
from jax.experimental.pallas import tpu as pltpu
from jax.experimental.pallas import tpu_sc as plsc

N_NODES = 4096
N_EDGES = 131072
HID = 128
H2 = 256

_NC = 2
_NS = 16


def _ln(x, g, b, eps=1e-5):
    m = jnp.mean(x, axis=-1, keepdims=True)
    v = jnp.mean((x - m) ** 2, axis=-1, keepdims=True)
    return (x - m) * lax.rsqrt(v + eps) * g + b


def _silu(x):
    return x / (1.0 + jnp.exp(-x))


def _dot(a, b):
    return jnp.dot(a, b, preferred_element_type=jnp.float32)


_B = 512
_EPC = N_EDGES // _NC
_NCH = _EPC // _B


@functools.lru_cache(maxsize=1)
def _build_sc_msg():
  mesh = plsc.VectorSubcoreMesh(
      core_axis_name="c", subcore_axis_name="s",
      num_cores=_NC, num_subcores=_NS)

  @functools.partial(
    pl.kernel,
    out_type=jax.ShapeDtypeStruct((_NC, N_NODES, H2), jnp.float32),
    mesh=mesh,
    compiler_params=pltpu.CompilerParams(
        needs_layout_passes=False, use_tc_tiling_on_sc=False),
    scratch_types=[
        pltpu.VMEM((N_NODES, 16), jnp.float32),
        pltpu.VMEM((2, _B), jnp.int32),
        pltpu.VMEM((2, _B), jnp.int32),
        pltpu.VMEM((2, _B), jnp.int32),
        pltpu.VMEM((2, _B), jnp.int32),
        pltpu.VMEM((2, _B, 16), jnp.float32),
        pltpu.VMEM((2, _B, 16), jnp.float32),
        pltpu.VMEM((2, _B, 16), jnp.float32),
        pltpu.SemaphoreType.DMA,
        pltpu.SemaphoreType.DMA,
        pltpu.SemaphoreType.DMA,
        pltpu.SemaphoreType.DMA,
        pltpu.SemaphoreType.DMA,
        pltpu.SemaphoreType.DMA,
        pltpu.SemaphoreType.DMA,
        pltpu.SemaphoreType.DMA,
        pltpu.SemaphoreType.DMA,
        pltpu.SemaphoreType.DMA,
    ],
  )
  def _sc_msg(dst_hbm, src_hbm, xd_hbm, xs_hbm, ep_hbm, out_hbm,
              acc, di, si, dib, sib, bd, bs, bp,
              sd0, ss0, sp0, sd1, ss1, sp1, si0, si1, si2, si3):
    cid = lax.axis_index("c")
    sid = lax.axis_index("s")

    def _zero(r, _):
        acc[r, :] = jnp.zeros((16,), jnp.float32)
        return 0
    lax.fori_loop(0, N_NODES, _zero, 0)

    base = pl.multiple_of(cid * _EPC, 8)
    iota16 = lax.iota(jnp.int32, 16)
    sems = ((sd0, ss0, sp0), (sd1, ss1, sp1))
    isems = ((si0, si1), (si2, si3))

    def _idx_descs(slot, i):
        eb = pl.multiple_of(base + i * _B, 8)
        sa, sb = isems[slot]
        return (
            pltpu.make_async_copy(dst_hbm.at[pl.ds(eb, _B)], di.at[slot], sa),
            pltpu.make_async_copy(src_hbm.at[pl.ds(eb, _B)], si.at[slot], sb),
        )

    def _descs(slot, i):
        eb = pl.multiple_of(base + i * _B, 8)
        sd, ss, sp = sems[slot]
        return (
            pltpu.make_async_copy(xd_hbm.at[dib.at[slot]], bd.at[slot], sd),
            pltpu.make_async_copy(xs_hbm.at[sib.at[slot]], bs.at[slot], ss),
            pltpu.make_async_copy(
                ep_hbm.at[pl.ds(eb, _B), sid, :], bp.at[slot], sp),
        )

    def _start_idx(slot, i):
        for d in _idx_descs(slot, i):
            d.start()

    def _launch(slot, i):
        for d in _idx_descs(slot, i):
            d.wait()

        @plsc.parallel_loop(0, _B // 16, 1, unroll=4)
        def _mkidx(b):
            sl = pl.ds(b * 16, 16)
            dib[slot, sl] = di[slot, sl] * 16 + sid
            sib[slot, sl] = si[slot, sl] * 16 + sid
        for d in _descs(slot, i):
            d.start()

    def _wait(slot, i):
        for d in _descs(slot, i):
            d.wait()

    def _compute(slot):
        slotv = jnp.full((16,), slot, jnp.int32)

        @plsc.parallel_loop(0, _B, 1, unroll=12)
        def _edge(e):
            a = bd[slot, e, :] + bs[slot, e, :] + bp[slot, e, :]
            sg = 1.0 / (1.0 + jnp.exp(-a))
            row = plsc.load_gather(di, [slotv, jnp.broadcast_to(e, (16,))])
            plsc.addupdate_scatter(acc, [row, iota16], a * sg)

    _start_idx(0, 0)
    _launch(0, 0)
    _start_idx(1, 1)

    def _step(slot, i):
        other = 1 - slot
        _wait(slot, i)
        _launch(other, i + 1)
        _compute(slot)

        @pl.when(i + 2 < _NCH)
        def _():
            _start_idx(slot, i + 2)

    def _pair(g, _):
        i0 = g * 2
        _step(0, i0)
        _step(1, i0 + 1)
        return 0

    lax.fori_loop(0, _NCH // 2 - 1, _pair, 0)
    _step(0, _NCH - 2)
    _wait(1, _NCH - 1)
    _compute(1)
    pltpu.sync_copy(acc, out_hbm.at[cid, :, pl.ds(sid * 16, 16)])

  return _sc_msg


def _sc_partials(dst, src, xd, xs, ep):
    xdv = xd.reshape(N_NODES * 16, 16)
    xsv = xs.reshape(N_NODES * 16, 16)
    return _build_sc_msg()(dst, src, xdv, xsv, ep.reshape(N_EDGES, 16, 16))


def _prelude_body(t_s, pe, tw1, tb1, tg1, tlb1, tw2, tb2, tg2, tlb2,
                  nf, new, neb, wd, ws, mb, x_out, xd, xs):
    row = pe[pl.ds(t_s[0], 1), :]
    h = _dot(row, tw1[...]) + tb1[...]
    h = _silu(_ln(h, tg1[...], tlb1[...]))
    h = _dot(h, tw2[...]) + tb2[...]
    temb = _ln(h, tg2[...], tlb2[...])
    xv = _dot(nf[...], new[...]) + neb[...] + temb
    x_out[...] = xv
    xd[...] = _dot(xv, wd[...]) + mb[...]
    xs[...] = _dot(xv, ws[...])


def _edge_body(ef, wee, bee, w0, w1, w2, w3, ew1, eb1, eg1, elb1, ew2, eb2,
               ep0, ep1, ep2, ep3, eno):
    ea = _dot(ef[...], wee[...]) + bee[...]
    n2 = 2 * ea.shape[0]
    ep0[...] = _dot(ea, w0[...]).reshape(n2, 128)
    ep1[...] = _dot(ea, w1[...]).reshape(n2, 128)
    ep2[...] = _dot(ea, w2[...]).reshape(n2, 128)
    ep3[...] = _dot(ea, w3[...]).reshape(n2, 128)
    h = _dot(ea, ew1[...]) + eb1[...]
    h = _silu(_ln(h, eg1[...], elb1[...]))
    eno[...] = _dot(h, ew2[...]) + eb2[...]


def _ldense_body(x, wd, ws, mb, xd, xs):
    xv = x[...]
    xd[...] = _dot(xv, wd[...]) + mb[...]
    xs[...] = _dot(xv, ws[...])


def _lpost_body(parts, ow, ob, g, b, xres, xo):
    agg = parts[0] + parts[1]
    o = _dot(agg, ow[...]) + ob[...]
    xo[...] = _ln(o, g[...], b[...]) + xres[...]


def _lfused_body(parts, ow, ob, g, b, xres, wd, ws, mb, xo, xd, xs):
    agg = parts[0] + parts[1]
    o = _dot(agg, ow[...]) + ob[...]
    xv = _ln(o, g[...], b[...]) + xres[...]
    xo[...] = xv
    xd[...] = _dot(xv, wd[...]) + mb[...]
    xs[...] = _dot(xv, ws[...])


_BQ = 512


def _attn_body(x_ref, wq, wk, wv, bq, bk, bv, o_ref, k_s, v_s):
    qb = pl.program_id(1)

    @pl.when(qb == 0)
    def _():
        xv = x_ref[...]
        k_s[...] = _dot(xv, wk[0]) + bk[0]
        v_s[...] = _dot(xv, wv[0]) + bv[0]

    q = _dot(x_ref[pl.ds(qb * _BQ, _BQ), :], wq[0]) + bq[0]
    s = lax.dot_general(q, k_s[...], (((1,), (1,)), ((), ())),
                        preferred_element_type=jnp.float32) * 0.25
    m = jnp.max(s, axis=-1, keepdims=True)
    e = jnp.exp(s - m)
    p = e / jnp.sum(e, axis=-1, keepdims=True)
    o_ref[0] = _dot(p, v_s[...])


def _final_body(o, x, wo, bo, w1, b1, g, bb, w2, b2, out):
    xg = _dot(o[0], wo[0])
    for h in range(1, 8):
        xg = xg + _dot(o[h], wo[h])
    x2 = x[...] + xg + bo[...]
    h = _dot(x2, w1[...]) + b1[...]
    h = _silu(_ln(h, g[...], bb[...]))
    out[...] = _dot(h, w2[...]) + b2[...]


def _r2(v):
    return v.reshape(1, -1)


def kernel(node_features, edge_index, edge_features, t, params):
    p = params
    src = edge_index[0]
    dst = edge_index[1]

    f32 = jnp.float32
    vm = pl.BlockSpec(memory_space=pltpu.VMEM)

    layers = p['layers']
    wd_t = [lp['msg_w'][:, :HID].T for lp in layers]
    ws_t = [lp['msg_w'][:, HID:2 * HID].T for lp in layers]

    x, xd, xs = pl.pallas_call(
        _prelude_body,
        out_shape=[jax.ShapeDtypeStruct((N_NODES, HID), f32),
                   jax.ShapeDtypeStruct((N_NODES, H2), f32),
                   jax.ShapeDtypeStruct((N_NODES, H2), f32)],
        in_specs=[pl.BlockSpec(memory_space=pltpu.SMEM)] + [vm] * 15,
        out_specs=[vm] * 3,
    )(t, p['pe'],
      p['te_w1'].T, _r2(p['te_b1']), _r2(p['te_ln1_g']), _r2(p['te_ln1_b']),
      p['te_w2'].T, _r2(p['te_b2']), _r2(p['te_ln2_g']), _r2(p['te_ln2_b']),
      node_features, p['node_embed_w'].T, _r2(p['node_embed_b']),
      wd_t[0], ws_t[0], _r2(layers[0]['msg_b']))

    BE = 2048
    n_eb = N_EDGES // BE
    we_t = [lp['msg_w'][:, 2 * HID:].T for lp in p['layers']]
    const = lambda shp: pl.BlockSpec(shp, lambda i: tuple(0 for _ in shp))
    eouts = pl.pallas_call(
        _edge_body,
        grid=(n_eb,),
        out_shape=[jax.ShapeDtypeStruct((2 * N_EDGES, 128), f32)] * 4
        + [jax.ShapeDtypeStruct((N_EDGES, 16), f32)],
        in_specs=[pl.BlockSpec((BE, 16), lambda i: (i, 0)),
                  const((16, HID)), const((1, HID)),
                  const((HID, H2)), const((HID, H2)),
                  const((HID, H2)), const((HID, H2)),
                  const((HID, H2)), const((1, H2)), const((1, H2)),
                  const((1, H2)), const((H2, 16)), const((1, 16))],
        out_specs=[pl.BlockSpec((2 * BE, 128), lambda i: (i, 0))] * 4
        + [pl.BlockSpec((BE, 16), lambda i: (i, 0))],
    )(edge_features, p['edge_embed_w'].T, _r2(p['edge_embed_b']),
      we_t[0], we_t[1], we_t[2], we_t[3],
      p['edge_out_w1'].T, _r2(p['edge_out_b1']),
      _r2(p['edge_out_ln_g']), _r2(p['edge_out_ln_b']),
      p['edge_out_w2'].T, _r2(p['edge_out_b2']))
    ep_layers, edge_noise = eouts[:4], eouts[4]

    for li, lp in enumerate(layers):
        parts = _sc_partials(dst, src, xd, xs, ep_layers[li])

        if li + 1 < len(layers):
            nxt = layers[li + 1]
            x, xd, xs = pl.pallas_call(
                _lfused_body,
                out_shape=[jax.ShapeDtypeStruct((N_NODES, HID), f32),
                           jax.ShapeDtypeStruct((N_NODES, H2), f32),
                           jax.ShapeDtypeStruct((N_NODES, H2), f32)],
                in_specs=[vm] * 9,
                out_specs=[vm] * 3,
            )(parts, lp['out_w'].T, _r2(lp['out_b']),
              _r2(lp['ln_g']), _r2(lp['ln_b']), x,
              wd_t[li + 1], ws_t[li + 1], _r2(nxt['msg_b']))
        else:
            x = pl.pallas_call(
                _lpost_body,
                out_shape=jax.ShapeDtypeStruct((N_NODES, HID), f32),
                in_specs=[vm] * 6,
                out_specs=vm,
            )(parts, lp['out_w'].T, _r2(lp['out_b']),
              _r2(lp['ln_g']), _r2(lp['ln_b']), x)

    hd = HID // 8
    wh = lambda w: w.T.reshape(HID, 8, hd).transpose(1, 0, 2)
    bh = lambda b: b.reshape(8, 1, hd)
    o = pl.pallas_call(
        _attn_body,
        grid=(8, N_NODES // _BQ),
        out_shape=jax.ShapeDtypeStruct((8, N_NODES, hd), f32),
        in_specs=[pl.BlockSpec((N_NODES, HID), lambda h, q: (0, 0)),
                  pl.BlockSpec((1, HID, hd), lambda h, q: (h, 0, 0)),
                  pl.BlockSpec((1, HID, hd), lambda h, q: (h, 0, 0)),
                  pl.BlockSpec((1, HID, hd), lambda h, q: (h, 0, 0)),
                  pl.BlockSpec((1, 1, hd), lambda h, q: (h, 0, 0)),
                  pl.BlockSpec((1, 1, hd), lambda h, q: (h, 0, 0)),
                  pl.BlockSpec((1, 1, hd), lambda h, q: (h, 0, 0))],
        out_specs=pl.BlockSpec((1, _BQ, hd), lambda h, q: (h, q, 0)),
        scratch_shapes=[pltpu.VMEM((N_NODES, hd), f32),
                        pltpu.VMEM((N_NODES, hd), f32)],
    )(x, wh(p['attn_wq']), wh(p['attn_wk']), wh(p['attn_wv']),
      bh(p['attn_bq']), bh(p['attn_bk']), bh(p['attn_bv']))

    node_noise = pl.pallas_call(
        _final_body,
        out_shape=jax.ShapeDtypeStruct((N_NODES, HID), f32),
        in_specs=[vm] * 10,
        out_specs=vm,
    )(o, x, p['attn_wo'].T.reshape(8, hd, HID), _r2(p['attn_bo']),
      p['node_out_w1'].T, _r2(p['node_out_b1']),
      _r2(p['node_out_ln_g']), _r2(p['node_out_ln_b']),
      p['node_out_w2'].T, _r2(p['node_out_b2']))

    return (node_noise, edge_noise)


# speedup vs baseline: 1.0365x; 1.0365x over previous
import functools
import math

import jax
import jax.numpy as jnp
from jax import lax
from jax.experimental import pallas as pl
from jax.experimental.pallas import tpu as pltpu
from jax.experimental.pallas import tpu_sc as plsc

N_NODES = 4096
N_EDGES = 131072
HID = 128
H2 = 256

_NC = 2
_NS = 16


def _ln(x, g, b, eps=1e-5):
    m = jnp.mean(x, axis=-1, keepdims=True)
    v = jnp.mean((x - m) ** 2, axis=-1, keepdims=True)
    return (x - m) * lax.rsqrt(v + eps) * g + b


def _silu(x):
    return x / (1.0 + jnp.exp(-x))


def _dot(a, b):
    return jnp.dot(a, b, preferred_element_type=jnp.float32)


_B = 512
_EPC = N_EDGES // _NC
_NCH = _EPC // _B


@functools.lru_cache(maxsize=1)
def _build_sc_msg():
  mesh = plsc.VectorSubcoreMesh(
      core_axis_name="c", subcore_axis_name="s",
      num_cores=_NC, num_subcores=_NS)

  @functools.partial(
    pl.kernel,
    out_type=jax.ShapeDtypeStruct((_NC, N_NODES, H2), jnp.float32),
    mesh=mesh,
    compiler_params=pltpu.CompilerParams(
        needs_layout_passes=False, use_tc_tiling_on_sc=False),
    scratch_types=[
        pltpu.VMEM((N_NODES, 16), jnp.float32),
        pltpu.VMEM((2, _B), jnp.int32),
        pltpu.VMEM((2, _B), jnp.int32),
        pltpu.VMEM((2, _B), jnp.int32),
        pltpu.VMEM((2, _B), jnp.int32),
        pltpu.VMEM((2, _B, 16), jnp.float32),
        pltpu.VMEM((2, _B, 16), jnp.float32),
        pltpu.VMEM((2, _B, 16), jnp.float32),
        pltpu.SemaphoreType.DMA,
        pltpu.SemaphoreType.DMA,
        pltpu.SemaphoreType.DMA,
        pltpu.SemaphoreType.DMA,
        pltpu.SemaphoreType.DMA,
        pltpu.SemaphoreType.DMA,
        pltpu.SemaphoreType.DMA,
        pltpu.SemaphoreType.DMA,
        pltpu.SemaphoreType.DMA,
        pltpu.SemaphoreType.DMA,
    ],
  )
  def _sc_msg(dst_hbm, src_hbm, xd_hbm, xs_hbm, ep_hbm, out_hbm,
              acc, di, si, dib, sib, bd, bs, bp,
              sd0, ss0, sp0, sd1, ss1, sp1, si0, si1, si2, si3):
    cid = lax.axis_index("c")
    sid = lax.axis_index("s")

    def _zero(r, _):
        acc[r, :] = jnp.zeros((16,), jnp.float32)
        return 0
    lax.fori_loop(0, N_NODES, _zero, 0)

    base = pl.multiple_of(cid * _EPC, 8)
    iota16 = lax.iota(jnp.int32, 16)
    sems = ((sd0, ss0, sp0), (sd1, ss1, sp1))
    isems = ((si0, si1), (si2, si3))

    def _idx_descs(slot, i):
        eb = pl.multiple_of(base + i * _B, 8)
        sa, sb = isems[slot]
        return (
            pltpu.make_async_copy(dst_hbm.at[pl.ds(eb, _B)], di.at[slot], sa),
            pltpu.make_async_copy(src_hbm.at[pl.ds(eb, _B)], si.at[slot], sb),
        )

    def _descs(slot, i):
        eb = pl.multiple_of(base + i * _B, 8)
        sd, ss, sp = sems[slot]
        return (
            pltpu.make_async_copy(xd_hbm.at[dib.at[slot]], bd.at[slot], sd),
            pltpu.make_async_copy(xs_hbm.at[sib.at[slot]], bs.at[slot], ss),
            pltpu.make_async_copy(
                ep_hbm.at[pl.ds(eb, _B), sid, :], bp.at[slot], sp),
        )

    def _start_idx(slot, i):
        for d in _idx_descs(slot, i):
            d.start()

    def _launch(slot, i):
        for d in _idx_descs(slot, i):
            d.wait()

        @plsc.parallel_loop(0, _B // 16, 1, unroll=4)
        def _mkidx(b):
            sl = pl.ds(b * 16, 16)
            dib[slot, sl] = di[slot, sl] * 16 + sid
            sib[slot, sl] = si[slot, sl] * 16 + sid
        for d in _descs(slot, i):
            d.start()

    def _wait(slot, i):
        for d in _descs(slot, i):
            d.wait()

    def _compute(slot):
        slotv = jnp.full((16,), slot, jnp.int32)

        @plsc.parallel_loop(0, _B, 1, unroll=8)
        def _edge(e):
            a = bd[slot, e, :] + bs[slot, e, :] + bp[slot, e, :]
            sg = 1.0 / (1.0 + jnp.exp(-a))
            row = plsc.load_gather(di, [slotv, jnp.broadcast_to(e, (16,))])
            plsc.addupdate_scatter(acc, [row, iota16], a * sg)

    _start_idx(0, 0)
    _launch(0, 0)
    _start_idx(1, 1)

    def _step(slot, i):
        other = 1 - slot
        _wait(slot, i)
        _launch(other, i + 1)
        _compute(slot)

        @pl.when(i + 2 < _NCH)
        def _():
            _start_idx(slot, i + 2)

    def _pair(g, _):
        i0 = g * 2
        _step(0, i0)
        _step(1, i0 + 1)
        return 0

    lax.fori_loop(0, _NCH // 2 - 1, _pair, 0)
    _step(0, _NCH - 2)
    _wait(1, _NCH - 1)
    _compute(1)
    pltpu.sync_copy(acc, out_hbm.at[cid, :, pl.ds(sid * 16, 16)])

  return _sc_msg


def _sc_partials(dst, src, xd, xs, ep):
    xdv = xd.reshape(N_NODES * 16, 16)
    xsv = xs.reshape(N_NODES * 16, 16)
    return _build_sc_msg()(dst, src, xdv, xsv, ep.reshape(N_EDGES, 16, 16))


def _prelude_body(t_s, pe, tw1, tb1, tg1, tlb1, tw2, tb2, tg2, tlb2,
                  nf, new, neb, wd, ws, mb, x_out, xd, xs):
    row = pe[pl.ds(t_s[0], 1), :]
    h = _dot(row, tw1[...]) + tb1[...]
    h = _silu(_ln(h, tg1[...], tlb1[...]))
    h = _dot(h, tw2[...]) + tb2[...]
    temb = _ln(h, tg2[...], tlb2[...])
    xv = _dot(nf[...], new[...]) + neb[...] + temb
    x_out[...] = xv
    xd[...] = _dot(xv, wd[...]) + mb[...]
    xs[...] = _dot(xv, ws[...])


def _edge_body(ef, wee, bee, w0, w1, w2, w3, ew1, eb1, eg1, elb1, ew2, eb2,
               ep0, ep1, ep2, ep3, eno):
    ea = _dot(ef[...], wee[...]) + bee[...]
    n2 = 2 * ea.shape[0]
    ep0[...] = _dot(ea, w0[...]).reshape(n2, 128)
    ep1[...] = _dot(ea, w1[...]).reshape(n2, 128)
    ep2[...] = _dot(ea, w2[...]).reshape(n2, 128)
    ep3[...] = _dot(ea, w3[...]).reshape(n2, 128)
    h = _dot(ea, ew1[...]) + eb1[...]
    h = _silu(_ln(h, eg1[...], elb1[...]))
    eno[...] = _dot(h, ew2[...]) + eb2[...]


def _ldense_body(x, wd, ws, mb, xd, xs):
    xv = x[...]
    xd[...] = _dot(xv, wd[...]) + mb[...]
    xs[...] = _dot(xv, ws[...])


def _lpost_body(parts, ow, ob, g, b, xres, xo):
    agg = parts[0] + parts[1]
    o = _dot(agg, ow[...]) + ob[...]
    xo[...] = _ln(o, g[...], b[...]) + xres[...]


def _lfused_body(parts, ow, ob, g, b, xres, wd, ws, mb, xo, xd, xs):
    agg = parts[0] + parts[1]
    o = _dot(agg, ow[...]) + ob[...]
    xv = _ln(o, g[...], b[...]) + xres[...]
    xo[...] = xv
    xd[...] = _dot(xv, wd[...]) + mb[...]
    xs[...] = _dot(xv, ws[...])


_BQ = 512


def _attn_body(x_ref, wq, wk, wv, bq, bk, bv, o_ref, k_s, v_s):
    qb = pl.program_id(1)

    @pl.when(qb == 0)
    def _():
        xv = x_ref[...]
        k_s[...] = _dot(xv, wk[0]) + bk[0]
        v_s[...] = _dot(xv, wv[0]) + bv[0]

    q = _dot(x_ref[pl.ds(qb * _BQ, _BQ), :], wq[0]) + bq[0]
    s = lax.dot_general(q, k_s[...], (((1,), (1,)), ((), ())),
                        preferred_element_type=jnp.float32) * 0.25
    m = jnp.max(s, axis=-1, keepdims=True)
    e = jnp.exp(s - m)
    p = e / jnp.sum(e, axis=-1, keepdims=True)
    o_ref[0] = _dot(p, v_s[...])


def _final_body(o, x, wo, bo, w1, b1, g, bb, w2, b2, out):
    xg = _dot(o[0], wo[0])
    for h in range(1, 8):
        xg = xg + _dot(o[h], wo[h])
    x2 = x[...] + xg + bo[...]
    h = _dot(x2, w1[...]) + b1[...]
    h = _silu(_ln(h, g[...], bb[...]))
    out[...] = _dot(h, w2[...]) + b2[...]


def _r2(v):
    return v.reshape(1, -1)


def kernel(node_features, edge_index, edge_features, t, params):
    p = params
    src = edge_index[0]
    dst = edge_index[1]

    f32 = jnp.float32
    vm = pl.BlockSpec(memory_space=pltpu.VMEM)

    layers = p['layers']
    wd_t = [lp['msg_w'][:, :HID].T for lp in layers]
    ws_t = [lp['msg_w'][:, HID:2 * HID].T for lp in layers]

    x, xd, xs = pl.pallas_call(
        _prelude_body,
        out_shape=[jax.ShapeDtypeStruct((N_NODES, HID), f32),
                   jax.ShapeDtypeStruct((N_NODES, H2), f32),
                   jax.ShapeDtypeStruct((N_NODES, H2), f32)],
        in_specs=[pl.BlockSpec(memory_space=pltpu.SMEM)] + [vm] * 15,
        out_specs=[vm] * 3,
    )(t, p['pe'],
      p['te_w1'].T, _r2(p['te_b1']), _r2(p['te_ln1_g']), _r2(p['te_ln1_b']),
      p['te_w2'].T, _r2(p['te_b2']), _r2(p['te_ln2_g']), _r2(p['te_ln2_b']),
      node_features, p['node_embed_w'].T, _r2(p['node_embed_b']),
      wd_t[0], ws_t[0], _r2(layers[0]['msg_b']))

    BE = 2048
    n_eb = N_EDGES // BE
    we_t = [lp['msg_w'][:, 2 * HID:].T for lp in p['layers']]
    const = lambda shp: pl.BlockSpec(shp, lambda i: tuple(0 for _ in shp))
    eouts = pl.pallas_call(
        _edge_body,
        grid=(n_eb,),
        out_shape=[jax.ShapeDtypeStruct((2 * N_EDGES, 128), f32)] * 4
        + [jax.ShapeDtypeStruct((N_EDGES, 16), f32)],
        in_specs=[pl.BlockSpec((BE, 16), lambda i: (i, 0)),
                  const((16, HID)), const((1, HID)),
                  const((HID, H2)), const((HID, H2)),
                  const((HID, H2)), const((HID, H2)),
                  const((HID, H2)), const((1, H2)), const((1, H2)),
                  const((1, H2)), const((H2, 16)), const((1, 16))],
        out_specs=[pl.BlockSpec((2 * BE, 128), lambda i: (i, 0))] * 4
        + [pl.BlockSpec((BE, 16), lambda i: (i, 0))],
    )(edge_features, p['edge_embed_w'].T, _r2(p['edge_embed_b']),
      we_t[0], we_t[1], we_t[2], we_t[3],
      p['edge_out_w1'].T, _r2(p['edge_out_b1']),
      _r2(p['edge_out_ln_g']), _r2(p['edge_out_ln_b']),
      p['edge_out_w2'].T, _r2(p['edge_out_b2']))
    ep_layers, edge_noise = eouts[:4], eouts[4]

    for li, lp in enumerate(layers):
        parts = _sc_partials(dst, src, xd, xs, ep_layers[li])

        if li + 1 < len(layers):
            nxt = layers[li + 1]
            x, xd, xs = pl.pallas_call(
                _lfused_body,
                out_shape=[jax.ShapeDtypeStruct((N_NODES, HID), f32),
                           jax.ShapeDtypeStruct((N_NODES, H2), f32),
                           jax.ShapeDtypeStruct((N_NODES, H2), f32)],
                in_specs=[vm] * 9,
                out_specs=[vm] * 3,
            )(parts, lp['out_w'].T, _r2(lp['out_b']),
              _r2(lp['ln_g']), _r2(lp['ln_b']), x,
              wd_t[li + 1], ws_t[li + 1], _r2(nxt['msg_b']))
        else:
            x = pl.pallas_call(
                _lpost_body,
                out_shape=jax.ShapeDtypeStruct((N_NODES, HID), f32),
                in_specs=[vm] * 6,
                out_specs=vm,
            )(parts, lp['out_w'].T, _r2(lp['out_b']),
              _r2(lp['ln_g']), _r2(lp['ln_b']), x)

    hd = HID // 8
    wh = lambda w: w.T.reshape(HID, 8, hd).transpose(1, 0, 2)
    bh = lambda b: b.reshape(8, 1, hd)
    o = pl.pallas_call(
        _attn_body,
        grid=(8, N_NODES // _BQ),
        out_shape=jax.ShapeDtypeStruct((8, N_NODES, hd), f32),
        in_specs=[pl.BlockSpec((N_NODES, HID), lambda h, q: (0, 0)),
                  pl.BlockSpec((1, HID, hd), lambda h, q: (h, 0, 0)),
                  pl.BlockSpec((1, HID, hd), lambda h, q: (h, 0, 0)),
                  pl.BlockSpec((1, HID, hd), lambda h, q: (h, 0, 0)),
                  pl.BlockSpec((1, 1, hd), lambda h, q: (h, 0, 0)),
                  pl.BlockSpec((1, 1, hd), lambda h, q: (h, 0, 0)),
                  pl.BlockSpec((1, 1, hd), lambda h, q: (h, 0, 0))],
        out_specs=pl.BlockSpec((1, _BQ, hd), lambda h, q: (h, q, 0)),
        scratch_shapes=[pltpu.VMEM((N_NODES, hd), f32),
                        pltpu.VMEM((N_NODES, hd), f32)],
    )(x, wh(p['attn_wq']), wh(p['attn_wk']), wh(p['attn_wv']),
      bh(p['attn_bq']), bh(p['attn_bk']), bh(p['attn_bv']))

    node_noise = pl.pallas_call(
        _final_body,
        out_shape=jax.ShapeDtypeStruct((N_NODES, HID), f32),
        in_specs=[vm] * 10,
        out_specs=vm,
    )(o, x, p['attn_wo'].T.reshape(8, hd, HID), _r2(p['attn_bo']),
      p['node_out_w1'].T, _r2(p['node_out_b1']),
      _r2(p['node_out_ln_g']), _r2(p['node_out_ln_b']),
      p['node_out_w2'].T, _r2(p['node_out_b2']))

    return (node_noise, edge_noise)
